# Initial kernel scaffold; baseline (speedup 1.0000x reference)
import jax
import jax.numpy as jnp
from jax.experimental import pallas as pl


def kernel(x_nchw, w1, b1, w2, b2):
    raise NotImplementedError("write your pallas kernel here")


# speedup vs baseline: 15.6460x; 15.6460x over previous
import jax
import jax.numpy as jnp
from jax.experimental import pallas as pl
from jax.experimental.pallas import tpu as pltpu

_BN_EPS = 1e-5
_C = 3


def _ssum(a):
    return jnp.sum(jnp.sum(a, axis=1, keepdims=True), axis=0, keepdims=True)


def _moments_kernel(x_ref, acc_ref):
    @pl.when(pl.program_id(0) == 0)
    def _():
        acc_ref[...] = jnp.zeros_like(acc_ref)

    hw = x_ref.shape[1] // _C
    x0 = jnp.maximum(x_ref[:, 0 * hw:1 * hw], 0.0)
    x1 = jnp.maximum(x_ref[:, 1 * hw:2 * hw], 0.0)
    x2 = jnp.maximum(x_ref[:, 2 * hw:3 * hw], 0.0)

    acc_ref[...] += jnp.concatenate(
        [
            _ssum(x0), _ssum(x1), _ssum(x2),
            _ssum(x0 * x0), _ssum(x1 * x1), _ssum(x2 * x2),
            _ssum(x0 * x1), _ssum(x1 * x2), _ssum(x0 * x2),
        ],
        axis=0,
    )


def _apply_kernel(p_ref, x_ref, o_ref):
    hw = x_ref.shape[1] // _C
    x0 = jnp.maximum(x_ref[:, 0 * hw:1 * hw], 0.0)
    x1 = jnp.maximum(x_ref[:, 1 * hw:2 * hw], 0.0)
    x2 = jnp.maximum(x_ref[:, 2 * hw:3 * hw], 0.0)

    p = p_ref
    t0 = jnp.maximum(p[0] * x0 + p[1] * x1 + p[2] * x2 + p[9], 0.0)
    t1 = jnp.maximum(p[3] * x0 + p[4] * x1 + p[5] * x2 + p[10], 0.0)
    t2 = jnp.maximum(p[6] * x0 + p[7] * x1 + p[8] * x2 + p[11], 0.0)

    o_ref[:, 0 * hw:1 * hw] = p[12] * t0 + p[13] * t1 + p[14] * t2 + p[21]
    o_ref[:, 1 * hw:2 * hw] = p[15] * t0 + p[16] * t1 + p[17] * t2 + p[22]
    o_ref[:, 2 * hw:3 * hw] = p[18] * t0 + p[19] * t1 + p[20] * t2 + p[23]


def kernel(x_nchw, w1, b1, w2, b2):
    N, c_in, H, W = x_nchw.shape
    assert c_in == _C
    HW = H * W
    P = N * HW

    x2d = x_nchw.reshape(N, _C * HW)

    B = 1
    for cand in (64, 32, 16, 8, 4, 2):
        if N % cand == 0:
            B = cand
            break
    grid = (N // B,)
    x_spec = pl.BlockSpec((B, _C * HW), lambda i: (i, 0))
    smem_spec = pl.BlockSpec(memory_space=pltpu.MemorySpace.SMEM)

    acc = pl.pallas_call(
        _moments_kernel,
        out_shape=jax.ShapeDtypeStruct((9, 1), jnp.float32),
        grid=grid,
        in_specs=[x_spec],
        out_specs=pl.BlockSpec((9, 1), lambda i: (0, 0)),
        compiler_params=pltpu.CompilerParams(
            dimension_semantics=("arbitrary",),
            vmem_limit_bytes=48 * 1024 * 1024),
        cost_estimate=pl.CostEstimate(
            flops=15 * P, transcendentals=0, bytes_accessed=4 * _C * P),
    )(x2d)

    s = acc[0:3, 0]
    dg = acc[3:6, 0]
    xg = acc[6:9, 0]
    m_mat = jnp.stack([
        jnp.stack([dg[0], xg[0], xg[2]]),
        jnp.stack([xg[0], dg[1], xg[1]]),
        jnp.stack([xg[2], xg[1], dg[2]]),
    ])

    w1s = w1 @ s
    sum_v = w1s + P * b1
    sum_v2 = jnp.sum((w1 @ m_mat) * w1, axis=1) + 2.0 * b1 * w1s + P * b1 * b1
    mean = sum_v / P
    var = jnp.maximum(sum_v2 / P - mean * mean, 0.0)
    inv = jax.lax.rsqrt(var + _BN_EPS)

    w22 = w2 @ w2
    b22 = w2 @ b2 + b2
    a_mat = w22 @ (inv[:, None] * w1)
    d = w22 @ (inv * b1 + 1.0 - mean * inv) + b22

    params = jnp.concatenate(
        [a_mat.reshape(-1), d, w2.reshape(-1), b2]).astype(jnp.float32)

    out2d = pl.pallas_call(
        _apply_kernel,
        out_shape=jax.ShapeDtypeStruct((N, _C * HW), jnp.float32),
        grid=grid,
        in_specs=[smem_spec, x_spec],
        out_specs=x_spec,
        compiler_params=pltpu.CompilerParams(
            dimension_semantics=("arbitrary",),
            vmem_limit_bytes=48 * 1024 * 1024),
        cost_estimate=pl.CostEstimate(
            flops=40 * P, transcendentals=0, bytes_accessed=8 * _C * P),
    )(params, x2d)

    return out2d.reshape(N, _C, H, W)
